# baseline (speedup 1.0000x reference)
import numpy as np
import jax
import jax.numpy as jnp
from jax.experimental import pallas as pl
from jax.experimental.pallas import tpu as pltpu

_DILATIONS = [1, 2, 1, 2]


def _fused_kernel(x_ref, g_ref, bias1_ref, w_out2_ref, b_out2_ref, out_ref):
    T, N, F = x_ref.shape[1], x_ref.shape[2], x_ref.shape[3]
    O, H = bias1_ref.shape
    s = [x_ref[0, t] for t in range(T)]
    for d in _DILATIONS:
        s = [s[i] + s[i + d] for i in range(len(s) - d)]
    s_cat = jnp.concatenate(s, axis=1).astype(jnp.bfloat16)
    pre = []
    for o in range(O):
        pre.append(jax.lax.dot(s_cat, g_ref[o],
                               preferred_element_type=jnp.float32))
    h = jnp.stack(pre, axis=0) + bias1_ref[...][:, None, :]
    h = jnp.maximum(h, 0.0).reshape(O * N, H)
    y = jax.lax.dot(h, w_out2_ref[...],
                    preferred_element_type=jnp.float32)
    y = y + b_out2_ref[...]
    out_ref[0] = y.reshape(O, N, F)


def kernel(inputs, W_in, b_in, W_out1, b_out1, W_out2, b_out2):
    B, T, N, F = inputs.shape
    H = W_in.shape[1]
    S, O = W_out1.shape
    Mt = W_out1.T * np.float32(0.5 ** len(_DILATIONS))
    G = (Mt[:, :, None, None] * W_in[None, None, :, :]).reshape(O, S * F, H)
    G = G.astype(jnp.bfloat16)
    beta = jnp.sum(W_out1, axis=0)
    bias1 = beta[:, None] * b_in[None, :] + b_out1[:, None]
    out = pl.pallas_call(
        _fused_kernel,
        grid=(B,),
        in_specs=[
            pl.BlockSpec((1, T, N, F), lambda b: (b, 0, 0, 0)),
            pl.BlockSpec((O, S * F, H), lambda b: (0, 0, 0)),
            pl.BlockSpec((O, H), lambda b: (0, 0)),
            pl.BlockSpec((H, F), lambda b: (0, 0)),
            pl.BlockSpec((1, F), lambda b: (0, 0)),
        ],
        out_specs=pl.BlockSpec((1, O, N, F), lambda b: (b, 0, 0, 0)),
        out_shape=jax.ShapeDtypeStruct((B, O, N, F), jnp.float32),
        compiler_params=pltpu.CompilerParams(
            dimension_semantics=("arbitrary",)),
    )(inputs, G, bias1, W_out2, b_out2.reshape(1, F))
    return out
